# baseline (speedup 1.0000x reference)
import jax
import jax.numpy as jnp
import numpy as np
from jax.experimental import pallas as pl

N = 50000
E = 800000
B = 1
IN_MLP = 128
HID_MLP = 128
OUT_MLP = 16
HID_GNN = 16
OUT_GNN = 32
HEADS = 4
TE = 64
NLAYERS = 2


def _seg_softmax(logits, seg, num):
    m = jax.ops.segment_max(logits, seg, num_segments=num)
    l = logits - m[seg]
    ex = jnp.exp(l)
    s = jax.ops.segment_sum(ex, seg, num_segments=num)
    return ex / (s[seg] + 1e-16)


def _tconv(p, xn, edge_index, edge_attr, oc):
    src = edge_index[0]
    dst = edge_index[1]
    n = xn.shape[0]
    H, C = HEADS, oc
    q = (xn @ p["q"]["W"] + p["q"]["b"]).reshape(n, H, C)
    k = (xn @ p["k"]["W"] + p["k"]["b"]).reshape(n, H, C)
    v = (xn @ p["v"]["W"] + p["v"]["b"]).reshape(n, H, C)
    e = (edge_attr @ p["e"]["W"]).reshape(-1, H, C)
    q_i = q[dst]
    k_j = k[src] + e
    v_j = v[src] + e
    alpha = (q_i * k_j).sum(-1) / jnp.sqrt(jnp.float32(C))
    alpha = _seg_softmax(alpha, dst, n)
    out = jax.ops.segment_sum(v_j * alpha[..., None], dst, num_segments=n)
    out = out.mean(axis=1)
    out = out + xn @ p["skip"]["W"] + p["skip"]["b"]
    return out


def _sinusoidal(t):
    half = TE // 2
    emb = jnp.log(10000.0) / (half - 1)
    freqs = jnp.exp(-emb * jnp.arange(half, dtype=jnp.float32))
    e = t[..., None] * freqs
    return jnp.concatenate([jnp.sin(e), jnp.cos(e)], axis=-1)


def _passthrough(x_ref, o_ref):
    o_ref[...] = x_ref[...]


def kernel(x, y, edge_index, edge_weight, t, params):
    xf = x.reshape(-1, IN_MLP)
    nmlp = len(params["mlp"])
    for i, p in enumerate(params["mlp"]):
        xf = xf @ p["W"] + p["b"]
        if i < nmlp - 1:
            xf = jax.nn.relu(xf)
    xf = xf.reshape(B, N, OUT_MLP)
    xf = jax.nn.relu(xf)
    te = _sinusoidal(t)
    te = te @ params["time"]["W"] + params["time"]["b"]
    te = te.reshape(N, OUT_GNN)
    te = jnp.moveaxis(te, -1, 0).reshape(-1, B, N)
    te = jnp.moveaxis(te, 0, -1).reshape(B, N, OUT_MLP, 2)
    xf = xf * te[..., 0] + te[..., 1]
    g = y
    ea = edge_weight[:, None]
    ocs = [HID_GNN] * (NLAYERS + 1) + [OUT_GNN]
    nconv = len(params["convs"])
    for i, p in enumerate(params["convs"]):
        g = _tconv(p, g, edge_index, ea, ocs[i])
        if i < nconv - 1:
            g = jax.nn.relu(g)
            bn = params["bns"][i]
            mu = g.mean(axis=0)
            var = g.var(axis=0)
            g = (g - mu) / jnp.sqrt(var + 1e-5) * bn["gamma"] + bn["beta"]
    ge = jnp.moveaxis(g, -1, 0).reshape(-1, B, N)
    ge = jnp.moveaxis(ge, 0, -1).reshape(B, N, OUT_MLP, 2)
    out = xf * ge[..., 0] + ge[..., 1]
    out = pl.pallas_call(
        _passthrough,
        out_shape=jax.ShapeDtypeStruct(out.shape, out.dtype),
    )(out)
    return out


# speedup vs baseline: 7.7828x; 7.7828x over previous
import functools
import math

import jax
import jax.numpy as jnp
import numpy as np
from jax import lax
from jax.experimental import pallas as pl
from jax.experimental.pallas import tpu as pltpu
from jax.experimental.pallas import tpu_sc as plsc

N = 50000
E = 800000
B = 1
IN_MLP = 128
HID_MLP = 128
OUT_MLP = 16
HID_GNN = 16
OUT_GNN = 32
HEADS = 4
TE = 64
NLAYERS = 2

NC = 2
NS = 16
NW = NC * NS
L = 16

EPW = 25600
EP = NW * EPW
CHA = 256
NCHUNKA = EPW // CHA
CHS = 256
NCHUNKS = EPW // CHS
CHB = 128
NCHUNKB = EPW // CHB
NPAD = 50176
RPS = NPAD // NS
ZROWS = 392

_SC_PARAMS = pltpu.CompilerParams(
    needs_layout_passes=False, use_tc_tiling_on_sc=False)
_MESH = dict(core_axis_name="c", subcore_axis_name="s")


def _sc_pass(P, HC, C, HG, mode):
  inv = 1.0 / math.sqrt(C)
  HGRP = HEADS // HG
  scale = {"s64": 1.0 / 64.0, "s4": 0.25, "fin": 1.0}[mode]
  use_m = mode != "s64"
  fin = mode == "fin"

  out_type = [jax.ShapeDtypeStruct((NC, NPAD, 2 * HEADS), jnp.float32)]
  if fin:
    out_type = [jax.ShapeDtypeStruct((HGRP, EP, HG), jnp.float32)] + out_type
  scratch = [
      pltpu.VMEM((CHA,), jnp.int32),
      pltpu.VMEM((CHA,), jnp.int32),
      pltpu.VMEM((CHA,), jnp.float32),
      pltpu.VMEM((16,), jnp.float32),
      pltpu.VMEM((CHA, P), jnp.float32),
      pltpu.VMEM((CHA, HC), jnp.float32),
      pltpu.VMEM((CHA, 2 * HEADS), jnp.float32),
      pltpu.VMEM_SHARED((NPAD, 2 * HEADS), jnp.float32),
      pltpu.SemaphoreType.DMA,
      pltpu.SemaphoreType.DMA,
  ]
  if use_m:
    scratch.insert(6, pltpu.VMEM((CHA, 16), jnp.float32))
    scratch.append(pltpu.SemaphoreType.DMA)
  if fin:
    scratch.insert(0, pltpu.VMEM((HGRP, CHA, HG), jnp.float32))

  @functools.partial(
      pl.kernel,
      mesh=plsc.VectorSubcoreMesh(**_MESH),
      compiler_params=_SC_PARAMS,
      out_type=out_type if fin else out_type[0],
      scratch_types=scratch,
  )
  def kern(*args):
    it = iter(args)
    qw_hbm = next(it); k_hbm = next(it); src_hbm = next(it)
    dst_hbm = next(it); ew_hbm = next(it); g0_hbm = next(it)
    m_hbm = next(it) if use_m else None
    zeros_hbm = next(it)
    t_out = next(it) if fin else None
    sb_out = next(it)
    tbuf = next(it) if fin else None
    src_v = next(it); dst_v = next(it); ew_v = next(it); g0_v = next(it)
    qbuf = next(it); kbuf = next(it)
    mbuf = next(it) if use_m else None
    sbuf = next(it); sb_sh = next(it)
    sem_q = next(it); sem_k = next(it)
    sem_m = next(it) if use_m else None

    c = lax.axis_index("c")
    s = lax.axis_index("s")
    wid = s * NC + c
    pltpu.sync_copy(g0_hbm, g0_v)

    for z in range(RPS // ZROWS):
      r0 = s * RPS + z * ZROWS
      pltpu.sync_copy(zeros_hbm.at[pl.ds(r0, ZROWS)],
                      sb_sh.at[pl.ds(r0, ZROWS)])
    plsc.subcore_barrier()

    def chunk_body(i, carry):
      base = wid * EPW + i * CHA
      pltpu.sync_copy(src_hbm.at[pl.ds(base, CHA)], src_v)
      pltpu.sync_copy(dst_hbm.at[pl.ds(base, CHA)], dst_v)
      pltpu.sync_copy(ew_hbm.at[pl.ds(base, CHA)], ew_v)
      cp_q = pltpu.async_copy(qw_hbm.at[dst_v], qbuf, sem_q)
      cp_k = pltpu.async_copy(k_hbm.at[src_v], kbuf, sem_k)
      cp_q.wait()
      cp_k.wait()
      if use_m:
        pltpu.async_copy(m_hbm.at[dst_v], mbuf, sem_m).wait()

      def grp_body(g, carry2):
        lanes = g * L + lax.iota(jnp.int32, L)
        ewv = ew_v[pl.ds(pl.multiple_of(g * L, L), L)]
        g0v = g0_v[...]
        for h in range(HEADS):
          def dot_body(cc, acc):
            col = jnp.full((L,), h * C, jnp.int32) + cc
            qv = plsc.load_gather(qbuf, [lanes, col])
            kv = plsc.load_gather(kbuf, [lanes, col])
            return acc + qv * kv
          acc = lax.fori_loop(0, C, dot_body, jnp.zeros((L,), jnp.float32))
          qwe = plsc.load_gather(
              qbuf, [lanes, jnp.full((L,), HC + h, jnp.int32)])
          lv = (acc + ewv * qwe) * inv - g0v
          if use_m:
            mv = plsc.load_gather(
                mbuf, [lanes, jnp.full((L,), h, jnp.int32)])
            lv = lv - mv
          tv = jnp.exp(lv * scale)
          plsc.store_scatter(sbuf, [lanes, jnp.full((L,), h, jnp.int32)], tv)
          if fin:
            plsc.store_scatter(
                tbuf,
                [jnp.full((L,), h // HG, jnp.int32), lanes,
                 jnp.full((L,), h % HG, jnp.int32)], tv)
            plsc.store_scatter(
                sbuf, [lanes, jnp.full((L,), HEADS + h, jnp.int32)],
                tv * ewv)
          else:
            plsc.store_scatter(
                sbuf, [lanes, jnp.full((L,), HEADS + h, jnp.int32)],
                jnp.zeros((L,), jnp.float32))
        return carry2

      lax.fori_loop(0, CHA // L, grp_body, 0)
      if fin:
        for hg in range(HGRP):
          pltpu.sync_copy(tbuf.at[hg], t_out.at[hg, pl.ds(base, CHA)])
      pltpu.sync_copy(sbuf, sb_sh.at[dst_v], add=True)
      return carry

    lax.fori_loop(0, NCHUNKA, chunk_body, 0)
    plsc.subcore_barrier()
    for z in range(RPS // ZROWS):
      r0 = s * RPS + z * ZROWS
      pltpu.sync_copy(sb_sh.at[pl.ds(r0, ZROWS)],
                      sb_out.at[c, pl.ds(r0, ZROWS)])

  return kern


def _sc_aggregate(C, HG):
  HGRP = HEADS // HG
  HGC = HG * C

  @functools.partial(
      pl.kernel,
      mesh=plsc.VectorSubcoreMesh(**_MESH),
      compiler_params=_SC_PARAMS,
      out_type=jax.ShapeDtypeStruct((NC, HGRP, NPAD, HGC), jnp.float32),
      scratch_types=[
          pltpu.VMEM((CHB,), jnp.int32),
          pltpu.VMEM((CHB,), jnp.int32),
          pltpu.VMEM((CHB,), jnp.int32),
          pltpu.VMEM((CHB, HG), jnp.float32),
          pltpu.VMEM((CHB, HGC), jnp.float32),
          pltpu.VMEM((CHB, HGC), jnp.float32),
          pltpu.VMEM_SHARED((NPAD, HGC), jnp.float32),
          pltpu.SemaphoreType.DMA,
      ],
  )
  def kern(vg_hbm, t_hbm, src_hbm, dst_hbm, zeros_hbm, a_out,
           src_v, dst_v, idx_v, tbuf, vbuf, vals, a_sh, sem_v):
    c = lax.axis_index("c")
    s = lax.axis_index("s")
    wid = s * NC + c

    for hg in range(HGRP):
      for z in range(RPS // ZROWS):
        r0 = s * RPS + z * ZROWS
        pltpu.sync_copy(zeros_hbm.at[pl.ds(r0, ZROWS)],
                        a_sh.at[pl.ds(r0, ZROWS)])
      plsc.subcore_barrier()

      def chunk_body(i, carry):
        base = wid * EPW + i * CHB
        pltpu.sync_copy(src_hbm.at[pl.ds(base, CHB)], src_v)
        pltpu.sync_copy(dst_hbm.at[pl.ds(base, CHB)], dst_v)
        pltpu.sync_copy(t_hbm.at[hg, pl.ds(base, CHB)], tbuf)

        def off_body(g, carry2):
          sl = pl.ds(pl.multiple_of(g * L, L), L)
          idx_v[sl] = src_v[sl] + hg * NPAD
          return carry2

        lax.fori_loop(0, CHB // L, off_body, 0)
        pltpu.async_copy(vg_hbm.at[idx_v], vbuf, sem_v).wait()

        def mul_body(g, carry2):
          lanes = g * L + lax.iota(jnp.int32, L)
          for hh in range(HG):
            tv = plsc.load_gather(
                tbuf, [lanes, jnp.full((L,), hh, jnp.int32)])

            def col_body(cc, carry3):
              col = jnp.full((L,), hh * C, jnp.int32) + cc
              vv = plsc.load_gather(vbuf, [lanes, col])
              plsc.store_scatter(vals, [lanes, col], tv * vv)
              return carry3

            lax.fori_loop(0, C, col_body, 0)
          return carry2

        lax.fori_loop(0, CHB // L, mul_body, 0)
        pltpu.sync_copy(vals, a_sh.at[dst_v], add=True)
        return carry

      lax.fori_loop(0, NCHUNKB, chunk_body, 0)
      plsc.subcore_barrier()
      for z in range(RPS // ZROWS):
        r0 = s * RPS + z * ZROWS
        pltpu.sync_copy(a_sh.at[pl.ds(r0, ZROWS)],
                        a_out.at[c, hg, pl.ds(r0, ZROWS)])
      plsc.subcore_barrier()

  return kern


def _pad_width(HC):
  return ((HC + HEADS + L - 1) // L) * L


def _tconv_sc(p, g, src_p, dst_p, ew_p, oc):
  n = g.shape[0]
  H, C = HEADS, oc
  HC = H * C
  HG = 2 if C == 16 else 1
  HGRP = H // HG
  P = _pad_width(HC)

  q = g @ p["q"]["W"] + p["q"]["b"]
  k = g @ p["k"]["W"] + p["k"]["b"]
  v = g @ p["v"]["W"] + p["v"]["b"]
  We = p["e"]["W"][0]
  qh = q.reshape(n, H, C)
  qWe = (qh * We.reshape(H, C)).sum(-1)

  inv = 1.0 / math.sqrt(C)
  qn = jnp.sqrt((qh * qh).sum(-1))
  kn = jnp.sqrt((k.reshape(n, H, C) ** 2).sum(-1))
  wn = jnp.sqrt((We.reshape(H, C) ** 2).sum(-1))
  kmax = kn.max(0) + wn * jnp.max(jnp.abs(ew_p))
  g0 = jnp.max(qn * kmax) * inv
  g0_arr = jnp.full((16,), 1.0, jnp.float32) * g0

  qw = jnp.zeros((NPAD, P), jnp.float32)
  qw = qw.at[:n, :HC].set(q).at[:n, HC:HC + H].set(qWe)
  kk = jnp.zeros((NPAD, HC), jnp.float32).at[:n].set(k)
  vg = jnp.zeros((HGRP, NPAD, HG * C), jnp.float32)
  vg = vg.at[:, :n].set(v.reshape(n, HGRP, HG * C).transpose(1, 0, 2))
  vg = vg.reshape(HGRP * NPAD, HG * C)

  zeros_sb = jnp.zeros((NPAD, 2 * HEADS), jnp.float32)
  zeros_a = jnp.zeros((NPAD, HG * C), jnp.float32)
  common = (qw, kk, src_p, dst_p, ew_p, g0_arr)

  s64 = _sc_pass(P, HC, C, HG, "s64")(*common, zeros_sb).sum(0)[:, :HEADS]
  m1 = 64.0 * jnp.log(s64)
  m1_row = jnp.zeros((NPAD, 16), jnp.float32).at[:, :HEADS].set(m1)

  s4 = _sc_pass(P, HC, C, HG, "s4")(
      *common, m1_row, zeros_sb).sum(0)[:, :HEADS]
  m2 = m1 + 4.0 * jnp.log(s4)
  m2_row = jnp.zeros((NPAD, 16), jnp.float32).at[:, :HEADS].set(m2)

  t_arr, sb_parts = _sc_pass(P, HC, C, HG, "fin")(
      *common, m2_row, zeros_sb)
  a_parts = _sc_aggregate(C, HG)(vg, t_arr, src_p, dst_p, zeros_a)

  sb = sb_parts.sum(0)[:n]
  ssum = sb[:, :H]
  bsum = sb[:, H:]
  A = a_parts.sum(0)[:, :n]
  A = A.reshape(HGRP, n, HG, C).transpose(1, 0, 2, 3).reshape(n, H, C)
  o = (A + bsum[..., None] * We.reshape(H, C)) / (ssum + 1e-16)[..., None]
  out = o.mean(axis=1)
  out = out + g @ p["skip"]["W"] + p["skip"]["b"]
  return out


def _sinusoidal(t):
  half = TE // 2
  emb = jnp.log(10000.0) / (half - 1)
  freqs = jnp.exp(-emb * jnp.arange(half, dtype=jnp.float32))
  e = t[..., None] * freqs
  return jnp.concatenate([jnp.sin(e), jnp.cos(e)], axis=-1)


BN_MLP = 2000


def _mlp_film_kernel(x_ref, te_ref, w1, b1, w2, b2, w3, b3, w4, b4,
                     wts, bts, wtb, btb, o_ref):
  h = jnp.dot(x_ref[...], w1[...], preferred_element_type=jnp.float32)
  h = jax.nn.relu(h + b1[...])
  h = jnp.dot(h, w2[...], preferred_element_type=jnp.float32)
  h = jax.nn.relu(h + b2[...])
  h = jnp.dot(h, w3[...], preferred_element_type=jnp.float32)
  h = jax.nn.relu(h + b3[...])
  h = jnp.dot(h, w4[...], preferred_element_type=jnp.float32)
  h = jax.nn.relu(h + b4[...])
  scale = jnp.dot(te_ref[...], wts[...],
                  preferred_element_type=jnp.float32) + bts[...]
  shift = jnp.dot(te_ref[...], wtb[...],
                  preferred_element_type=jnp.float32) + btb[...]
  o_ref[...] = h * scale + shift


def _mlp_film(x2d, te_in, params):
  mlp = params["mlp"]
  wt = params["time"]["W"]
  bt = params["time"]["b"]
  wts, wtb = wt[:, 0::2], wt[:, 1::2]
  bts, btb = bt[0::2], bt[1::2]
  row = lambda v: v.reshape(1, -1)
  grid = N // BN_MLP
  const = lambda shape: pl.BlockSpec(shape, lambda i: (0, 0))
  return pl.pallas_call(
      _mlp_film_kernel,
      grid=(grid,),
      in_specs=[
          pl.BlockSpec((BN_MLP, IN_MLP), lambda i: (i, 0)),
          pl.BlockSpec((BN_MLP, TE), lambda i: (i, 0)),
          const((IN_MLP, HID_MLP)), const((1, HID_MLP)),
          const((HID_MLP, HID_MLP)), const((1, HID_MLP)),
          const((HID_MLP, HID_MLP)), const((1, HID_MLP)),
          const((HID_MLP, OUT_MLP)), const((1, OUT_MLP)),
          const((TE, OUT_MLP)), const((1, OUT_MLP)),
          const((TE, OUT_MLP)), const((1, OUT_MLP)),
      ],
      out_specs=pl.BlockSpec((BN_MLP, OUT_MLP), lambda i: (i, 0)),
      out_shape=jax.ShapeDtypeStruct((N, OUT_MLP), jnp.float32),
  )(x2d, te_in, mlp[0]["W"], row(mlp[0]["b"]), mlp[1]["W"], row(mlp[1]["b"]),
    mlp[2]["W"], row(mlp[2]["b"]), mlp[3]["W"], row(mlp[3]["b"]),
    wts, row(bts), wtb, row(btb))


def kernel(x, y, edge_index, edge_weight, t, params):
  te_in = _sinusoidal(t).reshape(N, TE)
  xf = _mlp_film(x.reshape(-1, IN_MLP), te_in, params)
  xf = xf.reshape(B, N, OUT_MLP)

  src = edge_index[0].astype(jnp.int32)
  dst = edge_index[1].astype(jnp.int32)
  padi = jnp.full((EP - E,), N, jnp.int32)
  src_p = jnp.concatenate([src, padi])
  dst_p = jnp.concatenate([dst, padi])
  ew_p = jnp.concatenate([edge_weight, jnp.zeros((EP - E,), jnp.float32)])

  g = y
  ocs = [HID_GNN] * (NLAYERS + 1) + [OUT_GNN]
  nconv = len(params["convs"])
  for i, p in enumerate(params["convs"]):
    g = _tconv_sc(p, g, src_p, dst_p, ew_p, ocs[i])
    if i < nconv - 1:
      g = jax.nn.relu(g)
      bn = params["bns"][i]
      mu = g.mean(axis=0)
      var = g.var(axis=0)
      g = (g - mu) / jnp.sqrt(var + 1e-5) * bn["gamma"] + bn["beta"]

  ge = g.reshape(N, OUT_MLP, 2)
  out = xf * ge[None, :, :, 0] + ge[None, :, :, 1]
  return out
